# baseline (speedup 1.0000x reference)
import functools

import jax
import jax.numpy as jnp
from jax import lax
from jax.experimental import pallas as pl
from jax.experimental.pallas import tpu as pltpu
from jax.experimental.pallas import tpu_sc as plsc

N = 10000
E = 320000
D = 128

NC = 2
NS = 16
NW = NC * NS
LANES = 16

CH = 128
EPAD = 327680
TCH = EPAD // CH
CPW = EPAD // (NW * CH)
K = 2
NH = 2
HCPW = CPW // NH
NPAD = 10112
RPS = NPAD // NS

_mesh = plsc.VectorSubcoreMesh(core_axis_name="c", subcore_axis_name="s")


def _deg_body(dst_hbm, zeros_hbm, out_hbm, dstv, ones_v, deg_sh, sem):
    c = lax.axis_index("c")
    s = lax.axis_index("s")
    wid = s * NC + c

    @pl.loop(0, CH // LANES)
    def _(i):
        ones_v[pl.ds(i * LANES, LANES)] = jnp.full((LANES,), 1.0, jnp.float32)

    @pl.when(s == 0)
    def _():
        pltpu.sync_copy(zeros_hbm, deg_sh)

    pltpu.sync_copy(dst_hbm.at[pl.ds(wid * CPW, CPW)], dstv)
    plsc.subcore_barrier()

    @pl.loop(0, CPW)
    def _(i):
        pltpu.async_copy(ones_v, deg_sh.at[dstv.at[i]], sem, add=True)

    @pl.loop(0, CPW)
    def _(i):
        pltpu.make_async_copy(ones_v, deg_sh.at[dstv.at[0]], sem).wait()

    plsc.subcore_barrier()

    @pl.when(s == 0)
    def _():
        pltpu.sync_copy(deg_sh, out_hbm.at[c])


def _msg_body(xs_hbm, src_hbm, dst_hbm, zeros_hbm, out_hbm,
              srcv, dstv, rows, acc_sh, gsem, ssem):
    c = lax.axis_index("c")
    s = lax.axis_index("s")
    wid = s * NC + c

    pltpu.sync_copy(zeros_hbm.at[pl.ds(s * RPS, RPS)],
                    acc_sh.at[pl.ds(s * RPS, RPS)])
    plsc.subcore_barrier()

    def gather(i, b):
        return pltpu.async_copy(xs_hbm.at[srcv.at[i]], rows.at[b], gsem.at[b])

    def wait_gather(b):
        pltpu.make_async_copy(xs_hbm.at[srcv.at[0]], rows.at[b],
                              gsem.at[b]).wait()

    def scatter(i, b):
        return pltpu.async_copy(rows.at[b], acc_sh.at[dstv.at[i]],
                                ssem.at[b], add=True)

    def wait_scatter(b):
        pltpu.make_async_copy(rows.at[b], acc_sh.at[dstv.at[0]],
                              ssem.at[b]).wait()

    for h in range(NH):
        pltpu.sync_copy(src_hbm.at[pl.ds(wid * CPW + h * HCPW, HCPW)], srcv)
        pltpu.sync_copy(dst_hbm.at[pl.ds(wid * CPW + h * HCPW, HCPW)], dstv)
        for b in range(K):
            gather(b, b)

        @pl.loop(0, HCPW // K - 1)
        def _(g):
            for b in range(K):
                wait_gather(b)
                scatter(g * K + b, b)
            for b in range(K):
                wait_scatter(b)
                gather((g + 1) * K + b, b)

        for b in range(K):
            wait_gather(b)
            scatter(HCPW - K + b, b)
        for b in range(K):
            wait_scatter(b)

    plsc.subcore_barrier()
    pltpu.sync_copy(acc_sh.at[pl.ds(s * RPS, RPS)],
                    out_hbm.at[c, pl.ds(s * RPS, RPS)])


@functools.partial(pl.kernel,
                   out_type=jax.ShapeDtypeStruct((NC, NPAD), jnp.float32),
                   mesh=_mesh,
                   scratch_types=[
                       pltpu.VMEM((CPW, CH), jnp.int32),
                       pltpu.VMEM((CH,), jnp.float32),
                       pltpu.VMEM_SHARED((NPAD,), jnp.float32),
                       pltpu.SemaphoreType.DMA,
                   ])
def _deg_kernel(dst_hbm, zeros_hbm, out_hbm, dstv, ones_v, deg_sh, sem):
    _deg_body(dst_hbm, zeros_hbm, out_hbm, dstv, ones_v, deg_sh, sem)


@functools.partial(pl.kernel,
                   out_type=jax.ShapeDtypeStruct((NC, NPAD, D), jnp.float32),
                   mesh=_mesh,
                   scratch_types=[
                       pltpu.VMEM((HCPW, CH), jnp.int32),
                       pltpu.VMEM((HCPW, CH), jnp.int32),
                       pltpu.VMEM((K, CH, D), jnp.float32),
                       pltpu.VMEM_SHARED((NPAD, D), jnp.float32),
                       pltpu.SemaphoreType.DMA((K,)),
                       pltpu.SemaphoreType.DMA((K,)),
                   ])
def _msg_kernel(xs_hbm, src_hbm, dst_hbm, zeros_hbm, out_hbm,
                srcv, dstv, rows, acc_sh, gsem, ssem):
    _msg_body(xs_hbm, src_hbm, dst_hbm, zeros_hbm, out_hbm,
              srcv, dstv, rows, acc_sh, gsem, ssem)


def _xs_body(degt_ref, x_ref, xs_ref, dis_ref):
    deg = degt_ref[:N, 0:1] + degt_ref[:N, 1:2] + 1.0
    dis = lax.rsqrt(deg)
    dis_ref[...] = dis
    xs_ref[...] = x_ref[...] * dis


def _final_body(acc_ref, xs_ref, dis_ref, w_ref, gamma_ref, beta_ref, out_ref):
    a = acc_ref[0, :N, :] + acc_ref[1, :N, :] + xs_ref[...]
    z = a * dis_ref[...]
    y = jnp.dot(z, w_ref[...], preferred_element_type=jnp.float32)
    mean = jnp.mean(y, axis=0, keepdims=True)
    var = jnp.mean((y - mean) ** 2, axis=0, keepdims=True)
    out = (y - mean) * lax.rsqrt(var + 1e-5) * gamma_ref[...] + beta_ref[...]
    out_ref[...] = jnp.maximum(out, 0.0)


def kernel(x, edge_index, W, b, gamma, beta):
    del b
    ei = edge_index.astype(jnp.int32)
    pad = EPAD - E
    ar = jnp.arange(pad, dtype=jnp.int32)
    src = jnp.concatenate([ei[0], ar % N]).reshape(TCH, CH)
    dst = jnp.concatenate([ei[1], N + ar % (NPAD - N)]).reshape(TCH, CH)
    zeros1 = jnp.zeros((NPAD,), jnp.float32)
    zeros2 = jnp.zeros((NPAD, D), jnp.float32)

    degp = _deg_kernel(dst, zeros1)
    degt = jnp.transpose(degp)

    xs, dis = pl.pallas_call(
        _xs_body,
        out_shape=[jax.ShapeDtypeStruct((N, D), jnp.float32),
                   jax.ShapeDtypeStruct((N, 1), jnp.float32)],
    )(degt, x)

    acc = _msg_kernel(xs, src, dst, zeros2)

    out = pl.pallas_call(
        _final_body,
        out_shape=jax.ShapeDtypeStruct((N, D), jnp.float32),
    )(acc, xs, dis, W, gamma.reshape(1, D), beta.reshape(1, D))
    return out


# speedup vs baseline: 1.1284x; 1.1284x over previous
import functools

import jax
import jax.numpy as jnp
from jax import lax
from jax.experimental import pallas as pl
from jax.experimental.pallas import tpu as pltpu
from jax.experimental.pallas import tpu_sc as plsc

N = 10000
E = 320000
D = 128

NC = 2
NS = 16
NW = NC * NS
LANES = 16

CH = 64
EPAD = 327680
TCH = EPAD // CH
CPW = EPAD // (NW * CH)
K = 4
NH = 4
HCPW = CPW // NH
NPAD = 10112
RPS = NPAD // NS

_mesh = plsc.VectorSubcoreMesh(core_axis_name="c", subcore_axis_name="s")


def _deg_body(dst_hbm, zeros_hbm, out_hbm, dstv, ones_v, deg_sh, sem):
    c = lax.axis_index("c")
    s = lax.axis_index("s")
    wid = s * NC + c

    @pl.loop(0, CH // LANES)
    def _(i):
        ones_v[pl.ds(i * LANES, LANES)] = jnp.full((LANES,), 1.0, jnp.float32)

    @pl.when(s == 0)
    def _():
        pltpu.sync_copy(zeros_hbm, deg_sh)

    pltpu.sync_copy(dst_hbm.at[pl.ds(wid * CPW, CPW)], dstv)
    plsc.subcore_barrier()

    @pl.loop(0, CPW)
    def _(i):
        pltpu.async_copy(ones_v, deg_sh.at[dstv.at[i]], sem, add=True)

    @pl.loop(0, CPW)
    def _(i):
        pltpu.make_async_copy(ones_v, deg_sh.at[dstv.at[0]], sem).wait()

    plsc.subcore_barrier()

    @pl.when(s == 0)
    def _():
        pltpu.sync_copy(deg_sh, out_hbm.at[c])


def _msg_body(xs_hbm, src_hbm, dst_hbm, zeros_hbm, out_hbm,
              srcv, dstv, rows, acc_sh, gsem, ssem):
    c = lax.axis_index("c")
    s = lax.axis_index("s")
    wid = s * NC + c

    pltpu.sync_copy(zeros_hbm.at[pl.ds(s * RPS, RPS)],
                    acc_sh.at[pl.ds(s * RPS, RPS)])
    plsc.subcore_barrier()

    def gather(i, b):
        return pltpu.async_copy(xs_hbm.at[srcv.at[i]], rows.at[b], gsem.at[b])

    def wait_gather(b):
        pltpu.make_async_copy(xs_hbm.at[srcv.at[0]], rows.at[b],
                              gsem.at[b]).wait()

    def scatter(i, b):
        return pltpu.async_copy(rows.at[b], acc_sh.at[dstv.at[i]],
                                ssem.at[b], add=True)

    def wait_scatter(b):
        pltpu.make_async_copy(rows.at[b], acc_sh.at[dstv.at[0]],
                              ssem.at[b]).wait()

    for h in range(NH):
        pltpu.sync_copy(src_hbm.at[pl.ds(wid * CPW + h * HCPW, HCPW)], srcv)
        pltpu.sync_copy(dst_hbm.at[pl.ds(wid * CPW + h * HCPW, HCPW)], dstv)
        for b in range(K):
            gather(b, b)

        @pl.loop(0, HCPW // K - 1)
        def _(g):
            for b in range(K):
                wait_gather(b)
                scatter(g * K + b, b)
            for b in range(K):
                wait_scatter(b)
                gather((g + 1) * K + b, b)

        for b in range(K):
            wait_gather(b)
            scatter(HCPW - K + b, b)
        for b in range(K):
            wait_scatter(b)

    plsc.subcore_barrier()
    pltpu.sync_copy(acc_sh.at[pl.ds(s * RPS, RPS)],
                    out_hbm.at[c, pl.ds(s * RPS, RPS)])


@functools.partial(pl.kernel,
                   out_type=jax.ShapeDtypeStruct((NC, NPAD), jnp.float32),
                   mesh=_mesh,
                   scratch_types=[
                       pltpu.VMEM((CPW, CH), jnp.int32),
                       pltpu.VMEM((CH,), jnp.float32),
                       pltpu.VMEM_SHARED((NPAD,), jnp.float32),
                       pltpu.SemaphoreType.DMA,
                   ])
def _deg_kernel(dst_hbm, zeros_hbm, out_hbm, dstv, ones_v, deg_sh, sem):
    _deg_body(dst_hbm, zeros_hbm, out_hbm, dstv, ones_v, deg_sh, sem)


@functools.partial(pl.kernel,
                   out_type=jax.ShapeDtypeStruct((NC, NPAD, D), jnp.float32),
                   mesh=_mesh,
                   scratch_types=[
                       pltpu.VMEM((HCPW, CH), jnp.int32),
                       pltpu.VMEM((HCPW, CH), jnp.int32),
                       pltpu.VMEM((K, CH, D), jnp.float32),
                       pltpu.VMEM_SHARED((NPAD, D), jnp.float32),
                       pltpu.SemaphoreType.DMA((K,)),
                       pltpu.SemaphoreType.DMA((K,)),
                   ])
def _msg_kernel(xs_hbm, src_hbm, dst_hbm, zeros_hbm, out_hbm,
                srcv, dstv, rows, acc_sh, gsem, ssem):
    _msg_body(xs_hbm, src_hbm, dst_hbm, zeros_hbm, out_hbm,
              srcv, dstv, rows, acc_sh, gsem, ssem)


def _xs_body(degt_ref, x_ref, xs_ref, dis_ref):
    deg = degt_ref[:N, 0:1] + degt_ref[:N, 1:2] + 1.0
    dis = lax.rsqrt(deg)
    dis_ref[...] = dis
    xs_ref[...] = x_ref[...] * dis


def _final_body(acc_ref, xs_ref, dis_ref, w_ref, gamma_ref, beta_ref, out_ref):
    a = acc_ref[0, :N, :] + acc_ref[1, :N, :] + xs_ref[...]
    z = a * dis_ref[...]
    y = jnp.dot(z, w_ref[...], preferred_element_type=jnp.float32)
    mean = jnp.mean(y, axis=0, keepdims=True)
    var = jnp.mean((y - mean) ** 2, axis=0, keepdims=True)
    out = (y - mean) * lax.rsqrt(var + 1e-5) * gamma_ref[...] + beta_ref[...]
    out_ref[...] = jnp.maximum(out, 0.0)


def kernel(x, edge_index, W, b, gamma, beta):
    del b
    ei = edge_index.astype(jnp.int32)
    pad = EPAD - E
    ar = jnp.arange(pad, dtype=jnp.int32)
    src = jnp.concatenate([ei[0], ar % N]).reshape(TCH, CH)
    dst = jnp.concatenate([ei[1], N + ar % (NPAD - N)]).reshape(TCH, CH)
    zeros1 = jnp.zeros((NPAD,), jnp.float32)
    zeros2 = jnp.zeros((NPAD, D), jnp.float32)

    degp = _deg_kernel(dst, zeros1)
    degt = jnp.transpose(degp)

    xs, dis = pl.pallas_call(
        _xs_body,
        out_shape=[jax.ShapeDtypeStruct((N, D), jnp.float32),
                   jax.ShapeDtypeStruct((N, 1), jnp.float32)],
    )(degt, x)

    acc = _msg_kernel(xs, src, dst, zeros2)

    out = pl.pallas_call(
        _final_body,
        out_shape=jax.ShapeDtypeStruct((N, D), jnp.float32),
    )(acc, xs, dis, W, gamma.reshape(1, D), beta.reshape(1, D))
    return out
